# Initial kernel scaffold; baseline (speedup 1.0000x reference)
import jax
import jax.numpy as jnp
from jax.experimental import pallas as pl


def kernel(x, graph, params):
    raise NotImplementedError("write your pallas kernel here")


# speedup vs baseline: 2.9849x; 2.9849x over previous
import functools

import jax
import jax.numpy as jnp
from jax import lax
from jax.experimental import pallas as pl
from jax.experimental.pallas import tpu as pltpu
from jax.experimental.pallas import tpu_sc as plsc

N_NODES = 10000
N_EDGES = 320000
LATENT = 128


def _ln(h, g, bt):
    mu = jnp.mean(h, axis=-1, keepdims=True)
    var = jnp.mean((h - mu) * (h - mu), axis=-1, keepdims=True)
    return (h - mu) * lax.rsqrt(var + 1e-5) * g + bt


def _mlp2(xs, w1s, b1, g1, bt1, w2, b2, g2, bt2, *, block, final_act,
          l2norm=False):
    n_in = len(xs)
    rows = xs[0].shape[0]
    hidden = w2.shape[0]
    d_out = w2.shape[1]
    assert rows % block == 0

    def body(*refs):
        x_refs = refs[:n_in]
        w1_refs = refs[n_in:2 * n_in]
        b1_r, g1_r, bt1_r, w2_r, b2_r = refs[2 * n_in:2 * n_in + 5]
        rest = refs[2 * n_in + 5:]
        out_r = rest[-1]
        h = jnp.dot(x_refs[0][...], w1_refs[0][...],
                    preferred_element_type=jnp.float32)
        for i in range(1, n_in):
            h = h + jnp.dot(x_refs[i][...], w1_refs[i][...],
                            preferred_element_type=jnp.float32)
        h = h + b1_r[...]
        h = jax.nn.relu(_ln(h, g1_r[...], bt1_r[...]))
        o = jnp.dot(h, w2_r[...], preferred_element_type=jnp.float32) + b2_r[...]
        if final_act:
            g2_r, bt2_r = rest[0], rest[1]
            o = jax.nn.relu(_ln(o, g2_r[...], bt2_r[...]))
        if l2norm:
            nrm = jnp.sqrt(jnp.sum(o * o, axis=-1, keepdims=True))
            o = o / jnp.maximum(nrm, 1e-12)
        out_r[...] = o

    full = lambda shape: pl.BlockSpec(shape, lambda i: (0, 0))
    in_specs = (
        [pl.BlockSpec((block, x.shape[1]), lambda i: (i, 0)) for x in xs]
        + [full(w.shape) for w in w1s]
        + [full((1, hidden))] * 3
        + [full((hidden, d_out)), full((1, d_out))]
    )
    args = list(xs) + list(w1s) + [b1, g1, bt1, w2, b2]
    if final_act:
        in_specs += [full((1, d_out))] * 2
        args += [g2, bt2]
    return pl.pallas_call(
        body,
        grid=(rows // block,),
        in_specs=in_specs,
        out_specs=pl.BlockSpec((block, d_out), lambda i: (i, 0)),
        out_shape=jax.ShapeDtypeStruct((rows, d_out), jnp.float32),
        compiler_params=pltpu.CompilerParams(
            dimension_semantics=("arbitrary",)),
    )(*args)


def _mlp2_args(xs, w1s, p, *, block, final_act, l2norm=False):
    p0, p1 = p[0], p[1]
    return _mlp2(
        xs, w1s,
        p0["b"][None, :], p0["g"][None, :], p0["bt"][None, :],
        p1["w"], p1["b"][None, :],
        p1["g"][None, :] if final_act else None,
        p1["bt"][None, :] if final_act else None,
        block=block, final_act=final_act, l2norm=l2norm)


_CHUNK = 80


def _gather2(table, isrc, idst):
    info = plsc.get_sparse_core_info()
    nc, ns = info.num_cores, info.num_subcores
    nw = nc * ns
    per_w = N_EDGES // nw
    n_ch = per_w // _CHUNK
    mesh = plsc.VectorSubcoreMesh(core_axis_name="c", subcore_axis_name="s")
    out_sd = jax.ShapeDtypeStruct((N_EDGES, LATENT), jnp.float32)

    @functools.partial(
        pl.kernel,
        out_type=(out_sd, out_sd),
        mesh=mesh,
        scratch_types=[
            pltpu.VMEM((_CHUNK,), jnp.int32),
            pltpu.VMEM((_CHUNK, LATENT), jnp.float32),
            pltpu.VMEM((_CHUNK,), jnp.int32),
            pltpu.VMEM((_CHUNK, LATENT), jnp.float32),
            pltpu.SemaphoreType.DMA,
            pltpu.SemaphoreType.DMA,
        ],
    )
    def k(table_h, isrc_h, idst_h, osrc_h, odst_h, ia, ra, ib, rb, sa, sb):
        wid = lax.axis_index("s") * nc + lax.axis_index("c")
        base = wid * per_w

        def step(j, carry):
            off = base + j * _CHUNK
            pltpu.sync_copy(isrc_h.at[pl.ds(off, _CHUNK)], ia)
            cpa = pltpu.async_copy(table_h.at[ia], ra, sa)
            pltpu.sync_copy(idst_h.at[pl.ds(off, _CHUNK)], ib)
            cpb = pltpu.async_copy(table_h.at[ib], rb, sb)
            cpa.wait()
            pltpu.sync_copy(ra, osrc_h.at[pl.ds(off, _CHUNK)])
            cpb.wait()
            pltpu.sync_copy(rb, odst_h.at[pl.ds(off, _CHUNK)])
            return carry

        lax.fori_loop(0, n_ch, step, 0)

    return k(table, isrc, idst)


def _scatter_add(edges, idst, zeros):
    info = plsc.get_sparse_core_info()
    nc, ns = info.num_cores, info.num_subcores
    per_core = N_EDGES // nc
    per_tile = per_core // ns
    n_ch = per_tile // _CHUNK
    zb = 640
    zlast = N_NODES - zb * (ns - 1)
    mesh = plsc.VectorSubcoreMesh(core_axis_name="c", subcore_axis_name="s")

    @functools.partial(
        pl.kernel,
        out_type=jax.ShapeDtypeStruct((nc * N_NODES, LATENT), jnp.float32),
        mesh=mesh,
        scratch_types=[
            pltpu.VMEM((_CHUNK,), jnp.int32),
            pltpu.VMEM((_CHUNK, LATENT), jnp.float32),
            pltpu.VMEM_SHARED((N_NODES, LATENT), jnp.float32),
        ],
    )
    def k(edges_h, idst_h, zeros_h, out_h, idx_v, rows_v, acc):
        cid = lax.axis_index("c")
        sid = lax.axis_index("s")

        @pl.when(sid < ns - 1)
        def _():
            pltpu.sync_copy(zeros_h.at[pl.ds(sid * zb, zb)],
                            acc.at[pl.ds(sid * zb, zb)])

        @pl.when(sid == ns - 1)
        def _():
            pltpu.sync_copy(zeros_h.at[pl.ds(zb * (ns - 1), zlast)],
                            acc.at[pl.ds(zb * (ns - 1), zlast)])

        plsc.subcore_barrier()
        base = cid * per_core + sid * per_tile

        def step(j, carry):
            off = base + j * _CHUNK
            pltpu.sync_copy(idst_h.at[pl.ds(off, _CHUNK)], idx_v)
            pltpu.sync_copy(edges_h.at[pl.ds(off, _CHUNK)], rows_v)
            pltpu.sync_copy(rows_v, acc.at[idx_v], add=True)
            return carry

        lax.fori_loop(0, n_ch, step, 0)
        plsc.subcore_barrier()

        @pl.when(sid < ns - 1)
        def _():
            pltpu.sync_copy(acc.at[pl.ds(sid * zb, zb)],
                            out_h.at[pl.ds(cid * N_NODES + sid * zb, zb)])

        @pl.when(sid == ns - 1)
        def _():
            pltpu.sync_copy(
                acc.at[pl.ds(zb * (ns - 1), zlast)],
                out_h.at[pl.ds(cid * N_NODES + zb * (ns - 1), zlast)])

    return k(edges, idst, zeros)


def kernel(x, graph, params):
    src = graph[0]
    dst = graph[1]
    latent = LATENT

    ne = params["node_enc"]
    nodes = _mlp2_args([x], [ne[0]["w"]], ne, block=1000, final_act=True)

    gx_s, gx_d = _gather2(x, src, dst)
    ee = params["edge_enc"]
    w = ee[0]["w"]
    edges = _mlp2_args([gx_s, gx_d], [w[:latent], w[latent:]], ee,
                       block=1280, final_act=True)

    zeros = jnp.zeros((N_NODES, latent), jnp.float32)
    for cp in params["cells"]:
        gn_s, gn_d = _gather2(nodes, src, dst)
        we = cp["edge"][0]["w"]
        edges = _mlp2_args(
            [gn_s, gn_d, edges],
            [we[:latent], we[latent:2 * latent], we[2 * latent:]],
            cp["edge"], block=1280, final_act=True)
        msgs = _scatter_add(edges, dst, zeros)
        wn = cp["node"][0]["w"]
        nodes = _mlp2_args(
            [nodes, msgs[:N_NODES], msgs[N_NODES:]],
            [wn[:latent], wn[latent:], wn[latent:]],
            cp["node"], block=1000, final_act=True)

    po = params["out"]
    emb = _mlp2_args([nodes], [po[0]["w"]], po, block=1000, final_act=False,
                     l2norm=True)
    return emb, nodes, edges


# speedup vs baseline: 3.4467x; 1.1547x over previous
import functools

import jax
import jax.numpy as jnp
from jax import lax
from jax.experimental import pallas as pl
from jax.experimental.pallas import tpu as pltpu
from jax.experimental.pallas import tpu_sc as plsc

N_NODES = 10000
N_EDGES = 320000
LATENT = 128


def _ln(h, g, bt):
    mu = jnp.mean(h, axis=-1, keepdims=True)
    var = jnp.mean((h - mu) * (h - mu), axis=-1, keepdims=True)
    return (h - mu) / jnp.sqrt(var + 1e-5) * g + bt


def _mlp2(xs, w1s, b1, g1, bt1, w2, b2, g2, bt2, *, block, final_act,
          l2norm=False):
    n_in = len(xs)
    rows = xs[0].shape[0]
    hidden = w2.shape[0]
    d_out = w2.shape[1]
    assert rows % block == 0

    def body(*refs):
        x_refs = refs[:n_in]
        w1_refs = refs[n_in:2 * n_in]
        b1_r, g1_r, bt1_r, w2_r, b2_r = refs[2 * n_in:2 * n_in + 5]
        rest = refs[2 * n_in + 5:]
        out_r = rest[-1]
        h = jnp.dot(x_refs[0][...], w1_refs[0][...],
                    preferred_element_type=jnp.float32)
        for i in range(1, n_in):
            h = h + jnp.dot(x_refs[i][...], w1_refs[i][...],
                            preferred_element_type=jnp.float32)
        h = h + b1_r[...]
        h = jax.nn.relu(_ln(h, g1_r[...], bt1_r[...]))
        o = jnp.dot(h, w2_r[...], preferred_element_type=jnp.float32) + b2_r[...]
        if final_act:
            g2_r, bt2_r = rest[0], rest[1]
            o = jax.nn.relu(_ln(o, g2_r[...], bt2_r[...]))
        if l2norm:
            nrm = jnp.sqrt(jnp.sum(o * o, axis=-1, keepdims=True))
            o = o / jnp.maximum(nrm, 1e-12)
        out_r[...] = o

    full = lambda shape: pl.BlockSpec(shape, lambda i: (0, 0))
    in_specs = (
        [pl.BlockSpec((block, x.shape[1]), lambda i: (i, 0)) for x in xs]
        + [full(w.shape) for w in w1s]
        + [full((1, hidden))] * 3
        + [full((hidden, d_out)), full((1, d_out))]
    )
    args = list(xs) + list(w1s) + [b1, g1, bt1, w2, b2]
    if final_act:
        in_specs += [full((1, d_out))] * 2
        args += [g2, bt2]
    return pl.pallas_call(
        body,
        grid=(rows // block,),
        in_specs=in_specs,
        out_specs=pl.BlockSpec((block, d_out), lambda i: (i, 0)),
        out_shape=jax.ShapeDtypeStruct((rows, d_out), jnp.float32),
        compiler_params=pltpu.CompilerParams(
            dimension_semantics=("arbitrary",)),
    )(*args)


def _mlp2_args(xs, w1s, p, *, block, final_act, l2norm=False):
    p0, p1 = p[0], p[1]
    return _mlp2(
        xs, w1s,
        p0["b"][None, :], p0["g"][None, :], p0["bt"][None, :],
        p1["w"], p1["b"][None, :],
        p1["g"][None, :] if final_act else None,
        p1["bt"][None, :] if final_act else None,
        block=block, final_act=final_act, l2norm=l2norm)


_CHUNK = 80


def _gather2(table, isrc, idst):
    info = plsc.get_sparse_core_info()
    nc, ns = info.num_cores, info.num_subcores
    nw = nc * ns
    per_w = N_EDGES // nw
    n_ch = per_w // _CHUNK
    mesh = plsc.VectorSubcoreMesh(core_axis_name="c", subcore_axis_name="s")
    out_sd = jax.ShapeDtypeStruct((N_EDGES, LATENT), jnp.float32)

    assert n_ch % 2 == 1
    n_pair = n_ch // 2

    @functools.partial(
        pl.kernel,
        out_type=(out_sd, out_sd),
        mesh=mesh,
        scratch_types=(
            [pltpu.VMEM((_CHUNK,), jnp.int32)] * 4
            + [pltpu.VMEM((_CHUNK, LATENT), jnp.float32)] * 4
            + [pltpu.SemaphoreType.DMA] * 8
        ),
    )
    def k(table_h, isrc_h, idst_h, osrc_h, odst_h,
          ia0, ia1, ib0, ib1, ra0, ra1, rb0, rb1,
          sga0, sga1, sgb0, sgb1, swa0, swa1, swb0, swb1):
        ia = (ia0, ia1)
        ib = (ib0, ib1)
        ra = (ra0, ra1)
        rb = (rb0, rb1)
        sga = (sga0, sga1)
        sgb = (sgb0, sgb1)
        swa = (swa0, swa1)
        swb = (swb0, swb1)
        wid = lax.axis_index("s") * nc + lax.axis_index("c")
        base = wid * per_w

        def fetch(c, u, first):
            off = base + c * _CHUNK

            def drain():
                pltpu.make_async_copy(
                    ra[u], osrc_h.at[pl.ds(base, _CHUNK)], swa[u]).wait()
                pltpu.make_async_copy(
                    rb[u], odst_h.at[pl.ds(base, _CHUNK)], swb[u]).wait()

            if first is None:
                drain()
            else:
                pl.when(first)(drain)
            pltpu.sync_copy(isrc_h.at[pl.ds(off, _CHUNK)], ia[u])
            pltpu.sync_copy(idst_h.at[pl.ds(off, _CHUNK)], ib[u])
            pltpu.async_copy(table_h.at[ia[u]], ra[u], sga[u])
            pltpu.async_copy(table_h.at[ib[u]], rb[u], sgb[u])

        def flush(c, u):
            off = base + c * _CHUNK
            pltpu.make_async_copy(table_h.at[ia[u]], ra[u], sga[u]).wait()
            pltpu.async_copy(ra[u], osrc_h.at[pl.ds(off, _CHUNK)], swa[u])
            pltpu.make_async_copy(table_h.at[ib[u]], rb[u], sgb[u]).wait()
            pltpu.async_copy(rb[u], odst_h.at[pl.ds(off, _CHUNK)], swb[u])

        def pair(kk, carry):
            for u in range(2):
                fetch(2 * kk + u, u, kk > 0)
            for u in range(2):
                flush(2 * kk + u, u)
            return carry

        lax.fori_loop(0, n_pair, pair, 0)
        fetch(n_ch - 1, 0, None)
        flush(n_ch - 1, 0)
        for u in range(2):
            pltpu.make_async_copy(
                ra[u], osrc_h.at[pl.ds(base, _CHUNK)], swa[u]).wait()
            pltpu.make_async_copy(
                rb[u], odst_h.at[pl.ds(base, _CHUNK)], swb[u]).wait()

    return k(table, isrc, idst)


def _scatter_add(edges, idst, zeros):
    info = plsc.get_sparse_core_info()
    nc, ns = info.num_cores, info.num_subcores
    per_core = N_EDGES // nc
    per_tile = per_core // ns
    n_ch = per_tile // _CHUNK
    zb = 640
    zlast = N_NODES - zb * (ns - 1)
    mesh = plsc.VectorSubcoreMesh(core_axis_name="c", subcore_axis_name="s")

    assert n_ch % 2 == 1
    n_pair = n_ch // 2

    @functools.partial(
        pl.kernel,
        out_type=jax.ShapeDtypeStruct((nc * N_NODES, LATENT), jnp.float32),
        mesh=mesh,
        scratch_types=(
            [pltpu.VMEM((_CHUNK,), jnp.int32)] * 2
            + [pltpu.VMEM((_CHUNK, LATENT), jnp.float32)] * 2
            + [pltpu.VMEM_SHARED((N_NODES, LATENT), jnp.float32)]
            + [pltpu.SemaphoreType.DMA] * 4
        ),
    )
    def k(edges_h, idst_h, zeros_h, out_h, ix0, ix1, ev0, ev1, acc,
          sld0, sld1, ssc0, ssc1):
        ix = (ix0, ix1)
        ev = (ev0, ev1)
        sld = (sld0, sld1)
        ssc = (ssc0, ssc1)
        cid = lax.axis_index("c")
        sid = lax.axis_index("s")

        @pl.when(sid < ns - 1)
        def _():
            pltpu.sync_copy(zeros_h.at[pl.ds(sid * zb, zb)],
                            acc.at[pl.ds(sid * zb, zb)])

        @pl.when(sid == ns - 1)
        def _():
            pltpu.sync_copy(zeros_h.at[pl.ds(zb * (ns - 1), zlast)],
                            acc.at[pl.ds(zb * (ns - 1), zlast)])

        plsc.subcore_barrier()
        base = cid * per_core + sid * per_tile

        def fetch(c, u, nonfirst):
            off = base + c * _CHUNK

            def drain():
                pltpu.make_async_copy(ev[u], acc.at[ix[u]], ssc[u]).wait()

            if nonfirst is None:
                drain()
            else:
                pl.when(nonfirst)(drain)
            pltpu.sync_copy(idst_h.at[pl.ds(off, _CHUNK)], ix[u])
            pltpu.async_copy(edges_h.at[pl.ds(off, _CHUNK)], ev[u], sld[u])

        def flush(c, u):
            off = base + c * _CHUNK
            pltpu.make_async_copy(
                edges_h.at[pl.ds(off, _CHUNK)], ev[u], sld[u]).wait()
            pltpu.async_copy(ev[u], acc.at[ix[u]], ssc[u], add=True)

        def pair(kk, carry):
            for u in range(2):
                fetch(2 * kk + u, u, kk > 0)
            for u in range(2):
                flush(2 * kk + u, u)
            return carry

        lax.fori_loop(0, n_pair, pair, 0)
        fetch(n_ch - 1, 0, None)
        flush(n_ch - 1, 0)
        for u in range(2):
            pltpu.make_async_copy(ev[u], acc.at[ix[u]], ssc[u]).wait()
        plsc.subcore_barrier()

        @pl.when(sid < ns - 1)
        def _():
            pltpu.sync_copy(acc.at[pl.ds(sid * zb, zb)],
                            out_h.at[pl.ds(cid * N_NODES + sid * zb, zb)])

        @pl.when(sid == ns - 1)
        def _():
            pltpu.sync_copy(
                acc.at[pl.ds(zb * (ns - 1), zlast)],
                out_h.at[pl.ds(cid * N_NODES + zb * (ns - 1), zlast)])

    return k(edges, idst, zeros)


def kernel(x, graph, params):
    src = graph[0]
    dst = graph[1]
    latent = LATENT

    ne = params["node_enc"]
    nodes = _mlp2_args([x], [ne[0]["w"]], ne, block=1000, final_act=True)

    gx_s, gx_d = _gather2(x, src, dst)
    ee = params["edge_enc"]
    w = ee[0]["w"]
    edges = _mlp2_args([gx_s, gx_d], [w[:latent], w[latent:]], ee,
                       block=1280, final_act=True)

    zeros = jnp.zeros((N_NODES, latent), jnp.float32)
    for cp in params["cells"]:
        gn_s, gn_d = _gather2(nodes, src, dst)
        we = cp["edge"][0]["w"]
        edges = _mlp2_args(
            [gn_s, gn_d, edges],
            [we[:latent], we[latent:2 * latent], we[2 * latent:]],
            cp["edge"], block=1280, final_act=True)
        msgs = _scatter_add(edges, dst, zeros)
        wn = cp["node"][0]["w"]
        nodes = _mlp2_args(
            [nodes, msgs[:N_NODES], msgs[N_NODES:]],
            [wn[:latent], wn[latent:], wn[latent:]],
            cp["node"], block=1000, final_act=True)

    po = params["out"]
    emb = _mlp2_args([nodes], [po[0]["w"]], po, block=1000, final_act=False,
                     l2norm=True)
    return emb, nodes, edges
